# baseline (speedup 1.0000x reference)
import functools

import jax
import jax.numpy as jnp
from jax import lax
from jax.experimental import pallas as pl
from jax.experimental.pallas import tpu as pltpu
from jax.experimental.pallas import tpu_sc as plsc

_D = 256
_HD = 64
_NSUB = 16
_NCORE = 2
_CH = 80
_ZR = 64

_SC_PARAMS = pltpu.CompilerParams(use_tc_tiling_on_sc=False,
                                  needs_layout_passes=False)


def _mesh():
    return plsc.VectorSubcoreMesh(core_axis_name="c", subcore_axis_name="s",
                                  num_cores=_NCORE, num_subcores=_NSUB)


def _rsqrt_pos(x):
    i = lax.bitcast_convert_type(x, jnp.int32)
    y = lax.bitcast_convert_type(jnp.int32(0x5F3759DF) - (i >> 1),
                                 jnp.float32)
    for _ in range(3):
        y = y * (1.5 - 0.5 * x * y * y)
    return y


def _zero_acc(zd_v, acc_s, s, acc_rows):
    for kk in range(acc_rows // _NSUB // _ZR):
        pltpu.sync_copy(zd_v, acc_s.at[pl.ds(s * (acc_rows // _NSUB)
                                             + kk * _ZR, _ZR)])


def _prop_chunks(src_hbm, rowc_v, dstc_v, nrmc_v, rows, sidx, gsem, ssem,
                 acc_s, n_chunks, scale):
    def gather(k, b):
        pltpu.async_copy(src_hbm.at[rowc_v.at[pl.ds(k * _CH, _CH)]],
                         rows[b], gsem[b])

    def wait_scatter(b):
        pltpu.make_async_copy(rows[b], acc_s.at[sidx[b]], ssem[b]).wait()

    def step(k, cur, oth):
        off = k * _CH
        for g in range(_CH // 16):
            sidx[cur][pl.ds(g * 16, 16)] = dstc_v[pl.ds(off + g * 16, 16)]
        pltpu.make_async_copy(src_hbm.at[rowc_v.at[pl.ds(off, _CH)]],
                              rows[cur], gsem[cur]).wait()

        def rowmul(rr, _):
            nrm16 = nrmc_v[pl.ds(off + rr * 16, 16)] * scale
            for u in range(16):
                sc = nrm16[u]
                r = rr * 16 + u
                for j in range(_HD // 16):
                    sl = pl.ds(j * 16, 16)
                    rows[cur][r, sl] = rows[cur][r, sl] * sc
            return 0

        lax.fori_loop(0, _CH // 16, rowmul, 0)

        @pl.when(k >= 1)
        def _():
            wait_scatter(oth)

        @pl.when(k + 1 < n_chunks)
        def _():
            gather(k + 1, oth)

        pltpu.async_copy(rows[cur], acc_s.at[sidx[cur]], ssem[cur],
                         add=True)

    @pl.when(n_chunks > 0)
    def _():
        gather(0, 0)

    def chunk(k, _):
        @pl.when(k % 2 == 0)
        def _():
            step(k, 0, 1)

        @pl.when(k % 2 == 1)
        def _():
            step(k, 1, 0)

        return 0

    lax.fori_loop(0, n_chunks, chunk, 0)

    @pl.when(n_chunks > 0)
    def _():
        par = (n_chunks - 1) % 2

        @pl.when(par == 0)
        def _():
            wait_scatter(0)

        @pl.when(par == 1)
        def _():
            wait_scatter(1)


def _prop_passes(srcs, dsts, rowc_v, dstc_v, nrmc_v, rows, sidx, gsem,
                 ssem, zd_v, acc_s, s, c, acc_rows, n_chunks, half, scale,
                 first_pass_zeroed):
    for f, (src, dst) in enumerate(zip(srcs, dsts)):
        if f > 0 or not first_pass_zeroed:
            _zero_acc(zd_v, acc_s, s, acc_rows)
        plsc.subcore_barrier()
        _prop_chunks(src, rowc_v, dstc_v, nrmc_v, rows, sidx, gsem, ssem,
                     acc_s, n_chunks, scale)
        plsc.subcore_barrier()

        @pl.when(s == 0)
        def _():
            pltpu.sync_copy(acc_s.at[pl.ds(0, half)],
                            dst.at[pl.ds(c * half, half)])

        plsc.subcore_barrier()


@functools.lru_cache(maxsize=None)
def _spmm_first(n, e):
    half = n // 2
    acc_rows = -(-(half + 1) // (_NSUB * _ZR)) * (_NSUB * _ZR)
    ept = e // _NSUB
    eptp = ept + 16
    n_chunks = ept // _CH
    assert e % (_NSUB * _CH) == 0 and n % 16 == 0
    deg_rows = -(-n // (_NSUB * _ZR)) * (_NSUB * _ZR)
    dslice = deg_rows // _NSUB

    @functools.partial(
        pl.kernel,
        out_type=tuple([jax.ShapeDtypeStruct((n, _HD), jnp.float32)] * 4
                       + [jax.ShapeDtypeStruct((deg_rows,), jnp.float32)]),
        mesh=_mesh(),
        compiler_params=_SC_PARAMS,
        scratch_types=[
            pltpu.VMEM((ept,), jnp.int32),
            pltpu.VMEM((ept,), jnp.int32),
            pltpu.VMEM((ept,), jnp.float32),
            pltpu.VMEM((eptp,), jnp.int32),
            pltpu.VMEM((eptp,), jnp.int32),
            pltpu.VMEM((eptp,), jnp.float32),
            pltpu.VMEM((deg_rows,), jnp.float32),
            pltpu.VMEM((_CH, _HD), jnp.float32),
            pltpu.VMEM((_CH, _HD), jnp.float32),
            pltpu.VMEM((dslice,), jnp.float32),
            pltpu.VMEM((_CH,), jnp.int32),
            pltpu.VMEM((_CH,), jnp.int32),
            pltpu.VMEM((_ZR, _HD), jnp.float32),
            pltpu.VMEM((dslice,), jnp.float32),
            pltpu.SemaphoreType.DMA,
            pltpu.SemaphoreType.DMA,
            pltpu.SemaphoreType.DMA,
            pltpu.SemaphoreType.DMA,
            pltpu.VMEM_SHARED((acc_rows, _HD), jnp.float32),
            pltpu.VMEM_SHARED((deg_rows,), jnp.float32),
            pltpu.VMEM_SHARED((deg_rows,), jnp.float32),
        ],
    )
    def k(x0_hbm, x1_hbm, x2_hbm, x3_hbm, row_hbm, col_hbm, w_hbm,
          t10_hbm, t11_hbm, t12_hbm, t13_hbm, dis_hbm,
          row_v, col_v, w_v, rowc_v, dstc_v, nrmc_v, dd_v,
          rows0_v, rows1_v, flat_v, sidx0_v, sidx1_v, zd_v, z1_v,
          gsem0, gsem1, ssem0, ssem1, acc_s, deg_s, dis_s):
        rows = (rows0_v, rows1_v)
        sidx = (sidx0_v, sidx1_v)
        gsem = (gsem0, gsem1)
        ssem = (ssem0, ssem1)
        c = lax.axis_index("c")
        s = lax.axis_index("s")
        lo = c * half
        base = s * ept

        zf = jnp.zeros((16,), jnp.float32)

        def zrow(r, _):
            for j in range(_HD // 16):
                zd_v[r, pl.ds(j * 16, 16)] = zf
            return 0

        lax.fori_loop(0, _ZR, zrow, 0)

        def zflat(i, _):
            z1_v[pl.ds(i * 16, 16)] = zf
            return 0

        lax.fori_loop(0, dslice // 16, zflat, 0)

        _zero_acc(zd_v, acc_s, s, acc_rows)
        pltpu.sync_copy(z1_v, deg_s.at[pl.ds(s * dslice, dslice)])

        pltpu.sync_copy(row_hbm.at[pl.ds(base, ept)], row_v)
        pltpu.sync_copy(col_hbm.at[pl.ds(base, ept)], col_v)
        pltpu.sync_copy(w_hbm.at[pl.ds(base, ept)], w_v)

        plsc.subcore_barrier()

        def deg_chunk(kk, _):
            off = kk * _CH
            for g in range(_CH // 16):
                sidx0_v[pl.ds(g * 16, 16)] = row_v[pl.ds(off + g * 16, 16)]
            pltpu.sync_copy(w_v.at[pl.ds(off, _CH)], deg_s.at[sidx0_v],
                            add=True)
            return 0

        lax.fori_loop(0, n_chunks, deg_chunk, 0)
        plsc.subcore_barrier()

        pltpu.sync_copy(deg_s.at[pl.ds(s * dslice, dslice)], flat_v)

        def dis_blk(g, _):
            sl = pl.ds(g * 16, 16)
            d = flat_v[sl]
            m = d > 0.0
            y = _rsqrt_pos(jnp.where(m, d, 1.0))
            flat_v[sl] = jnp.where(m, y, 0.0)
            return 0

        lax.fori_loop(0, dslice // 16, dis_blk, 0)
        pltpu.sync_copy(flat_v, dis_s.at[pl.ds(s * dslice, dslice)])

        @pl.when(c == 0)
        def _():
            pltpu.sync_copy(flat_v, dis_hbm.at[pl.ds(s * dslice, dslice)])

        plsc.subcore_barrier()
        pltpu.sync_copy(dis_s, dd_v)

        zi = jnp.zeros((16,), jnp.int32)
        trash = jnp.full((16,), half, jnp.int32)

        def prefill(i, _):
            sl = pl.ds(i * 16, 16)
            rowc_v[sl] = zi
            dstc_v[sl] = trash
            nrmc_v[sl] = zf
            return 0

        lax.fori_loop(0, eptp // 16, prefill, 0)

        def nrm_cmp(i, cnt):
            sl = pl.ds(i * 16, 16)
            r16 = row_v[sl]
            c16 = col_v[sl]
            a = plsc.load_gather(dd_v, [r16])
            b = plsc.load_gather(dd_v, [c16])
            nv = -(a * w_v[sl] * b)
            m = (c16 >= lo) & (c16 < lo + half)
            at = pl.ds(cnt, 16)
            plsc.store_compressed(rowc_v.at[at], r16, mask=m)
            plsc.store_compressed(dstc_v.at[at], c16 - lo, mask=m)
            plsc.store_compressed(nrmc_v.at[at], nv, mask=m)
            pc = plsc.all_reduce_population_count(m)
            return cnt + pc[0]

        cnt = lax.fori_loop(0, ept // 16, nrm_cmp, jnp.int32(0))
        ncd = (cnt + (_CH - 1)) // _CH

        _prop_passes((x0_hbm, x1_hbm, x2_hbm, x3_hbm),
                     (t10_hbm, t11_hbm, t12_hbm, t13_hbm),
                     rowc_v, dstc_v, nrmc_v, rows, sidx, gsem, ssem,
                     zd_v, acc_s, s, c, acc_rows, ncd, half, 1.0,
                     first_pass_zeroed=True)

    return k


@functools.lru_cache(maxsize=None)
def _spmm_second(n, e):
    half = n // 2
    acc_rows = -(-(half + 1) // (_NSUB * _ZR)) * (_NSUB * _ZR)
    ept = e // _NSUB
    eptp = ept + 16

    deg_rows = -(-n // (_NSUB * _ZR)) * (_NSUB * _ZR)

    @functools.partial(
        pl.kernel,
        out_type=tuple([jax.ShapeDtypeStruct((n, _HD), jnp.float32)] * 4),
        mesh=_mesh(),
        compiler_params=_SC_PARAMS,
        scratch_types=[
            pltpu.VMEM((ept,), jnp.int32),
            pltpu.VMEM((ept,), jnp.int32),
            pltpu.VMEM((ept,), jnp.float32),
            pltpu.VMEM((eptp,), jnp.int32),
            pltpu.VMEM((eptp,), jnp.int32),
            pltpu.VMEM((eptp,), jnp.float32),
            pltpu.VMEM((deg_rows,), jnp.float32),
            pltpu.VMEM((_CH, _HD), jnp.float32),
            pltpu.VMEM((_CH, _HD), jnp.float32),
            pltpu.VMEM((_CH,), jnp.int32),
            pltpu.VMEM((_CH,), jnp.int32),
            pltpu.VMEM((_ZR, _HD), jnp.float32),
            pltpu.SemaphoreType.DMA,
            pltpu.SemaphoreType.DMA,
            pltpu.SemaphoreType.DMA,
            pltpu.SemaphoreType.DMA,
            pltpu.VMEM_SHARED((acc_rows, _HD), jnp.float32),
        ],
    )
    def k(t10_hbm, t11_hbm, t12_hbm, t13_hbm, row_hbm, col_hbm, w_hbm,
          dis_hbm, a20_hbm, a21_hbm, a22_hbm, a23_hbm,
          row_v, col_v, w_v, rowc_v, dstc_v, nrmc_v, dd_v,
          rows0_v, rows1_v, sidx0_v, sidx1_v, zd_v,
          gsem0, gsem1, ssem0, ssem1, acc_s):
        rows = (rows0_v, rows1_v)
        sidx = (sidx0_v, sidx1_v)
        gsem = (gsem0, gsem1)
        ssem = (ssem0, ssem1)
        c = lax.axis_index("c")
        s = lax.axis_index("s")
        lo = c * half
        base = s * ept

        zf = jnp.zeros((16,), jnp.float32)

        def zrow(r, _):
            for j in range(_HD // 16):
                zd_v[r, pl.ds(j * 16, 16)] = zf
            return 0

        lax.fori_loop(0, _ZR, zrow, 0)

        pltpu.sync_copy(row_hbm.at[pl.ds(base, ept)], row_v)
        pltpu.sync_copy(col_hbm.at[pl.ds(base, ept)], col_v)
        pltpu.sync_copy(w_hbm.at[pl.ds(base, ept)], w_v)
        pltpu.sync_copy(dis_hbm, dd_v)

        zi = jnp.zeros((16,), jnp.int32)
        trash = jnp.full((16,), half, jnp.int32)

        def prefill(i, _):
            sl = pl.ds(i * 16, 16)
            rowc_v[sl] = zi
            dstc_v[sl] = trash
            nrmc_v[sl] = zf
            return 0

        lax.fori_loop(0, eptp // 16, prefill, 0)

        def nrm_cmp(i, cnt):
            sl = pl.ds(i * 16, 16)
            r16 = row_v[sl]
            c16 = col_v[sl]
            a = plsc.load_gather(dd_v, [r16])
            b = plsc.load_gather(dd_v, [c16])
            nv = -(a * w_v[sl] * b)
            m = (c16 >= lo) & (c16 < lo + half)
            at = pl.ds(cnt, 16)
            plsc.store_compressed(rowc_v.at[at], r16, mask=m)
            plsc.store_compressed(dstc_v.at[at], c16 - lo, mask=m)
            plsc.store_compressed(nrmc_v.at[at], nv, mask=m)
            pc = plsc.all_reduce_population_count(m)
            return cnt + pc[0]

        cnt = lax.fori_loop(0, ept // 16, nrm_cmp, jnp.int32(0))
        ncd = (cnt + (_CH - 1)) // _CH

        _prop_passes((t10_hbm, t11_hbm, t12_hbm, t13_hbm),
                     (a20_hbm, a21_hbm, a22_hbm, a23_hbm),
                     rowc_v, dstc_v, nrmc_v, rows, sidx, gsem, ssem,
                     zd_v, acc_s, s, c, acc_rows, ncd, half, 2.0,
                     first_pass_zeroed=False)

    return k


@functools.lru_cache(maxsize=None)
def _head(n):
    blk = 1000

    def body(x_ref, t10_ref, t11_ref, t12_ref, t13_ref,
             a20_ref, a21_ref, a22_ref, a23_ref,
             w_ref, b_ref, cw_ref, cb_ref, h_ref, lg_ref, pr_ref):
        w0 = w_ref[0] - w_ref[2]
        w1 = w_ref[1]
        w2 = w_ref[2]
        acc = jnp.dot(x_ref[...], w0, preferred_element_type=jnp.float32)
        for q, t1_ref in enumerate((t10_ref, t11_ref, t12_ref, t13_ref)):
            acc = acc + jnp.dot(t1_ref[...], w1[q * _HD:(q + 1) * _HD],
                                preferred_element_type=jnp.float32)
        for q, a2_ref in enumerate((a20_ref, a21_ref, a22_ref, a23_ref)):
            acc = acc + jnp.dot(a2_ref[...], w2[q * _HD:(q + 1) * _HD],
                                preferred_element_type=jnp.float32)
        acc = acc + b_ref[...]
        h = jnp.maximum(acc, 0.0)
        h_ref[...] = h
        lg = jnp.sum(h * cw_ref[...], axis=1, keepdims=True) + cb_ref[...]
        lg_ref[...] = lg
        pr_ref[...] = 1.0 / (1.0 + jnp.exp(-lg))

    return pl.pallas_call(
        body,
        grid=(n // blk,),
        in_specs=[
            pl.BlockSpec((blk, _D), lambda i: (i, 0)),
        ] + [pl.BlockSpec((blk, _HD), lambda i: (i, 0))] * 8 + [
            pl.BlockSpec((3, _D, _D), lambda i: (0, 0, 0)),
            pl.BlockSpec((1, _D), lambda i: (0, 0)),
            pl.BlockSpec((1, _D), lambda i: (0, 0)),
            pl.BlockSpec((1, 1), lambda i: (0, 0)),
        ],
        out_specs=[
            pl.BlockSpec((blk, _D), lambda i: (i, 0)),
            pl.BlockSpec((blk, 1), lambda i: (i, 0)),
            pl.BlockSpec((blk, 1), lambda i: (i, 0)),
        ],
        out_shape=[
            jax.ShapeDtypeStruct((n, _D), jnp.float32),
            jax.ShapeDtypeStruct((n, 1), jnp.float32),
            jax.ShapeDtypeStruct((n, 1), jnp.float32),
        ],
    )


def kernel(x, e_index, e_weights, W, bias, cls_w, cls_b):
    n, d = x.shape
    e = e_index.shape[1]
    row = e_index[0]
    col = e_index[1]
    xs = [x[:, q * _HD:(q + 1) * _HD] for q in range(d // _HD)]
    *t1s, dis = _spmm_first(n, e)(*xs, row, col, e_weights)
    a2s = _spmm_second(n, e)(*t1s, row, col, e_weights, dis)
    h, lg, pr = _head(n)(x, *t1s, *a2s, W, bias.reshape(1, d),
                         cls_w, cls_b.reshape(1, 1))
    return (jnp.squeeze(pr, -1), h, lg)


# speedup vs baseline: 2.1019x; 2.1019x over previous
import functools

import jax
import jax.numpy as jnp
from jax import lax
from jax.experimental import pallas as pl
from jax.experimental.pallas import tpu as pltpu
from jax.experimental.pallas import tpu_sc as plsc

_D = 256
_NSUB = 16
_NCORE = 2
_CH = 80
_ZR = 32

_SC_PARAMS = pltpu.CompilerParams(use_tc_tiling_on_sc=False,
                                  needs_layout_passes=False)


def _mesh():
    return plsc.VectorSubcoreMesh(core_axis_name="c", subcore_axis_name="s",
                                  num_cores=_NCORE, num_subcores=_NSUB)


def _rsqrt_pos(x):
    i = lax.bitcast_convert_type(x, jnp.int32)
    y = lax.bitcast_convert_type(jnp.int32(0x5F3759DF) - (i >> 1),
                                 jnp.float32)
    for _ in range(3):
        y = y * (1.5 - 0.5 * x * y * y)
    return y


def _zero_acc(zd_v, acc_s, s, acc_rows):
    for kk in range(acc_rows // _NSUB // _ZR):
        pltpu.sync_copy(zd_v, acc_s.at[pl.ds(s * (acc_rows // _NSUB)
                                             + kk * _ZR, _ZR)])


def _norms(row_v, col_v, w_v, dd_v, nrm_v, ept):
    def nrm_blk(i, _):
        sl = pl.ds(i * 16, 16)
        a = plsc.load_gather(dd_v, [row_v[sl]])
        b = plsc.load_gather(dd_v, [col_v[sl]])
        nrm_v[sl] = -(a * w_v[sl] * b)
        return 0

    lax.fori_loop(0, ept // 16, nrm_blk, 0)


def _prop(src_hbm, row_v, col_v, nrm_v, rows, sidx, gsem, ssem,
          acc_s, n_chunks, lo, half, scale, width):
    def gather(k, b):
        pltpu.async_copy(src_hbm.at[row_v.at[pl.ds(k * _CH, _CH)]],
                         rows[b], gsem[b])

    def wait_scatter(b):
        pltpu.make_async_copy(rows[b], acc_s.at[sidx[b]], ssem[b]).wait()

    def step(k, cur, oth):
        off = k * _CH
        @pl.when(k >= 1)
        def _():
            wait_scatter(oth)

        @pl.when(k + 1 < n_chunks)
        def _():
            gather(k + 1, oth)

        for g in range(_CH // 16):
            c16 = col_v[pl.ds(off + g * 16, 16)]
            inh = (c16 >= lo) & (c16 < lo + half)
            sidx[cur][pl.ds(g * 16, 16)] = jnp.where(inh, c16 - lo, half)
        pltpu.make_async_copy(src_hbm.at[row_v.at[pl.ds(off, _CH)]],
                              rows[cur], gsem[cur]).wait()

        def rowmul(rr, _):
            nrm16 = nrm_v[pl.ds(off + rr * 16, 16)] * scale
            for u in range(16):
                sc = nrm16[u]
                r = rr * 16 + u
                for j in range(width // 16):
                    sl = pl.ds(j * 16, 16)
                    rows[cur][r, sl] = rows[cur][r, sl] * sc
            return 0

        lax.fori_loop(0, _CH // 16, rowmul, 0)
        pltpu.async_copy(rows[cur], acc_s.at[sidx[cur]], ssem[cur],
                        add=True)

    gather(0, 0)

    def chunk(k, _):
        @pl.when(k % 2 == 0)
        def _():
            step(k, 0, 1)

        @pl.when(k % 2 == 1)
        def _():
            step(k, 1, 0)

        return 0

    lax.fori_loop(0, n_chunks, chunk, 0)
    wait_scatter((n_chunks - 1) % 2)


@functools.lru_cache(maxsize=None)
def _spmm_first(n, e):
    half = n // 2
    acc_rows = -(-(half + 1) // (_NSUB * _ZR)) * (_NSUB * _ZR)
    ept = e // _NSUB
    eptp = ept + 16
    n_chunks = ept // _CH
    assert e % (_NSUB * _CH) == 0 and n % 16 == 0
    deg_rows = -(-n // (_NSUB * _ZR)) * (_NSUB * _ZR)
    dslice = deg_rows // _NSUB

    @functools.partial(
        pl.kernel,
        out_type=(jax.ShapeDtypeStruct((n, _D // 2), jnp.float32),
                  jax.ShapeDtypeStruct((n, _D // 2), jnp.float32),
                  jax.ShapeDtypeStruct((e,), jnp.float32)),
        mesh=_mesh(),
        compiler_params=_SC_PARAMS,
        scratch_types=[
            pltpu.VMEM((ept,), jnp.int32),
            pltpu.VMEM((ept,), jnp.int32),
            pltpu.VMEM((ept,), jnp.float32),
            pltpu.VMEM((ept,), jnp.float32),
            pltpu.VMEM((deg_rows,), jnp.float32),
            pltpu.VMEM((_CH, _D // 2), jnp.float32),
            pltpu.VMEM((_CH, _D // 2), jnp.float32),
            pltpu.VMEM((dslice,), jnp.float32),
            pltpu.VMEM((_CH,), jnp.int32),
            pltpu.VMEM((_CH,), jnp.int32),
            pltpu.VMEM((_ZR, _D // 2), jnp.float32),
            pltpu.VMEM((dslice,), jnp.float32),
            pltpu.SemaphoreType.DMA,
            pltpu.SemaphoreType.DMA,
            pltpu.SemaphoreType.DMA,
            pltpu.SemaphoreType.DMA,
            pltpu.VMEM_SHARED((acc_rows, _D // 2), jnp.float32),
            pltpu.VMEM_SHARED((deg_rows,), jnp.float32),
            pltpu.VMEM_SHARED((deg_rows,), jnp.float32),
        ],
    )
    def k(xlo_hbm, xhi_hbm, row_hbm, col_hbm, w_hbm,
          t1lo_hbm, t1hi_hbm, norm_hbm,
          row_v, col_v, w_v, nrm_v, dd_v,
          rows0_v, rows1_v, flat_v, sidx0_v, sidx1_v, zd_v, z1_v,
          gsem0, gsem1, ssem0, ssem1, acc_s, deg_s, dis_s):
        rows = (rows0_v, rows1_v)
        sidx = (sidx0_v, sidx1_v)
        gsem = (gsem0, gsem1)
        ssem = (ssem0, ssem1)
        c = lax.axis_index("c")
        s = lax.axis_index("s")
        lo = c * half
        base = s * ept

        zf = jnp.zeros((16,), jnp.float32)

        def zrow(r, _):
            for j in range(_D // 2 // 16):
                zd_v[r, pl.ds(j * 16, 16)] = zf
            return 0

        lax.fori_loop(0, _ZR, zrow, 0)

        def zflat(i, _):
            z1_v[pl.ds(i * 16, 16)] = zf
            return 0

        lax.fori_loop(0, dslice // 16, zflat, 0)

        _zero_acc(zd_v, acc_s, s, acc_rows)
        pltpu.sync_copy(z1_v, deg_s.at[pl.ds(s * dslice, dslice)])

        pltpu.sync_copy(row_hbm.at[pl.ds(base, ept)], row_v)
        pltpu.sync_copy(col_hbm.at[pl.ds(base, ept)], col_v)
        pltpu.sync_copy(w_hbm.at[pl.ds(base, ept)], w_v)

        plsc.subcore_barrier()

        def deg_chunk(kk, _):
            off = kk * _CH
            for g in range(_CH // 16):
                sidx0_v[pl.ds(g * 16, 16)] = row_v[pl.ds(off + g * 16, 16)]
            pltpu.sync_copy(w_v.at[pl.ds(off, _CH)], deg_s.at[sidx0_v],
                            add=True)
            return 0

        lax.fori_loop(0, n_chunks, deg_chunk, 0)
        plsc.subcore_barrier()

        pltpu.sync_copy(deg_s.at[pl.ds(s * dslice, dslice)], flat_v)

        def dis_blk(g, _):
            sl = pl.ds(g * 16, 16)
            d = flat_v[sl]
            m = d > 0.0
            y = _rsqrt_pos(jnp.where(m, d, 1.0))
            flat_v[sl] = jnp.where(m, y, 0.0)
            return 0

        lax.fori_loop(0, dslice // 16, dis_blk, 0)
        pltpu.sync_copy(flat_v, dis_s.at[pl.ds(s * dslice, dslice)])
        plsc.subcore_barrier()
        pltpu.sync_copy(dis_s, dd_v)

        _norms(row_v, col_v, w_v, dd_v, nrm_v, ept)

        @pl.when(c == 0)
        def _():
            pltpu.sync_copy(nrm_v, norm_hbm.at[pl.ds(base, ept)])

        for f, (srcp, dstp) in enumerate(((xlo_hbm, t1lo_hbm),
                                          (xhi_hbm, t1hi_hbm))):
            if f > 0:
                _zero_acc(zd_v, acc_s, s, acc_rows)
            plsc.subcore_barrier()
            _prop(srcp, row_v, col_v, nrm_v, rows, sidx, gsem, ssem,
                  acc_s, n_chunks, lo, half, 1.0, _D // 2)
            plsc.subcore_barrier()

            @pl.when(s == 0)
            def _():
                pltpu.sync_copy(acc_s.at[pl.ds(0, half)],
                                dstp.at[pl.ds(c * half, half)])

            plsc.subcore_barrier()

    return k


@functools.lru_cache(maxsize=None)
def _spmm_second(n, e):
    half = n // 2
    acc_rows = -(-(half + 1) // (_NSUB * _ZR)) * (_NSUB * _ZR)
    ept = e // _NSUB
    eptp = ept + 16
    n_chunks = ept // _CH
    deg_rows = -(-n // (_NSUB * _ZR)) * (_NSUB * _ZR)
    dslice = deg_rows // _NSUB

    @functools.partial(
        pl.kernel,
        out_type=(jax.ShapeDtypeStruct((n, _D // 2), jnp.float32),
                  jax.ShapeDtypeStruct((n, _D // 2), jnp.float32)),
        mesh=_mesh(),
        compiler_params=_SC_PARAMS,
        scratch_types=[
            pltpu.VMEM((ept,), jnp.int32),
            pltpu.VMEM((ept,), jnp.int32),
            pltpu.VMEM((ept,), jnp.float32),
            pltpu.VMEM((_CH, _D // 2), jnp.float32),
            pltpu.VMEM((_CH, _D // 2), jnp.float32),
            pltpu.VMEM((_CH,), jnp.int32),
            pltpu.VMEM((_CH,), jnp.int32),
            pltpu.VMEM((_ZR, _D // 2), jnp.float32),
            pltpu.SemaphoreType.DMA,
            pltpu.SemaphoreType.DMA,
            pltpu.SemaphoreType.DMA,
            pltpu.SemaphoreType.DMA,
            pltpu.VMEM_SHARED((acc_rows, _D // 2), jnp.float32),
        ],
    )
    def k(t1lo_hbm, t1hi_hbm, row_hbm, col_hbm, norm_hbm,
          a2lo_hbm, a2hi_hbm,
          row_v, col_v, nrm_v,
          rows0_v, rows1_v, sidx0_v, sidx1_v, zd_v,
          gsem0, gsem1, ssem0, ssem1, acc_s):
        rows = (rows0_v, rows1_v)
        sidx = (sidx0_v, sidx1_v)
        gsem = (gsem0, gsem1)
        ssem = (ssem0, ssem1)
        c = lax.axis_index("c")
        s = lax.axis_index("s")
        lo = c * half
        base = s * ept

        zf = jnp.zeros((16,), jnp.float32)

        def zrow(r, _):
            for j in range(_D // 2 // 16):
                zd_v[r, pl.ds(j * 16, 16)] = zf
            return 0

        lax.fori_loop(0, _ZR, zrow, 0)
        _zero_acc(zd_v, acc_s, s, acc_rows)

        pltpu.sync_copy(row_hbm.at[pl.ds(base, ept)], row_v)
        pltpu.sync_copy(col_hbm.at[pl.ds(base, ept)], col_v)
        pltpu.sync_copy(norm_hbm.at[pl.ds(base, ept)], nrm_v)

        for f, (srcp, dstp) in enumerate(((t1lo_hbm, a2lo_hbm),
                                          (t1hi_hbm, a2hi_hbm))):
            if f > 0:
                _zero_acc(zd_v, acc_s, s, acc_rows)
            plsc.subcore_barrier()
            _prop(srcp, row_v, col_v, nrm_v, rows, sidx, gsem,
                  ssem, acc_s, n_chunks, lo, half, 2.0, _D // 2)
            plsc.subcore_barrier()

            @pl.when(s == 0)
            def _():
                pltpu.sync_copy(acc_s.at[pl.ds(0, half)],
                                dstp.at[pl.ds(c * half, half)])

            plsc.subcore_barrier()

    return k


@functools.lru_cache(maxsize=None)
def _head(n):
    blk = 1000

    hw = _D // 2

    def body(x_ref, t1lo_ref, t1hi_ref, a2lo_ref, a2hi_ref,
             w_ref, b_ref, cw_ref, cb_ref, h_ref, lg_ref, pr_ref):
        w0 = w_ref[0] - w_ref[2]
        w1 = w_ref[1]
        w2 = w_ref[2]
        acc = jnp.dot(x_ref[...], w0, preferred_element_type=jnp.float32)
        acc = acc + jnp.dot(t1lo_ref[...], w1[:hw],
                            preferred_element_type=jnp.float32)
        acc = acc + jnp.dot(t1hi_ref[...], w1[hw:],
                            preferred_element_type=jnp.float32)
        acc = acc + jnp.dot(a2lo_ref[...], w2[:hw],
                            preferred_element_type=jnp.float32)
        acc = acc + jnp.dot(a2hi_ref[...], w2[hw:],
                            preferred_element_type=jnp.float32)
        acc = acc + b_ref[...]
        h = jnp.maximum(acc, 0.0)
        h_ref[...] = h
        lg = jnp.sum(h * cw_ref[...], axis=1, keepdims=True) + cb_ref[...]
        lg_ref[...] = lg
        pr_ref[...] = 1.0 / (1.0 + jnp.exp(-lg))

    return pl.pallas_call(
        body,
        grid=(n // blk,),
        in_specs=[
            pl.BlockSpec((blk, _D), lambda i: (i, 0)),
        ] + [pl.BlockSpec((blk, _D // 2), lambda i: (i, 0))] * 4 + [
            pl.BlockSpec((3, _D, _D), lambda i: (0, 0, 0)),
            pl.BlockSpec((1, _D), lambda i: (0, 0)),
            pl.BlockSpec((1, _D), lambda i: (0, 0)),
            pl.BlockSpec((1, 1), lambda i: (0, 0)),
        ],
        out_specs=[
            pl.BlockSpec((blk, _D), lambda i: (i, 0)),
            pl.BlockSpec((blk, 1), lambda i: (i, 0)),
            pl.BlockSpec((blk, 1), lambda i: (i, 0)),
        ],
        out_shape=[
            jax.ShapeDtypeStruct((n, _D), jnp.float32),
            jax.ShapeDtypeStruct((n, 1), jnp.float32),
            jax.ShapeDtypeStruct((n, 1), jnp.float32),
        ],
    )


def kernel(x, e_index, e_weights, W, bias, cls_w, cls_b):
    n, d = x.shape
    e = e_index.shape[1]
    row = e_index[0]
    col = e_index[1]
    x_lo = x[:, :d // 2]
    x_hi = x[:, d // 2:]
    t1lo, t1hi, nrm = _spmm_first(n, e)(x_lo, x_hi, row, col, e_weights)
    a2lo, a2hi = _spmm_second(n, e)(t1lo, t1hi, row, col, nrm)
    h, lg, pr = _head(n)(x, t1lo, t1hi, a2lo, a2hi, W, bias.reshape(1, d),
                         cls_w, cls_b.reshape(1, 1))
    return (jnp.squeeze(pr, -1), h, lg)
